# baseline (speedup 1.0000x reference)
import functools

import jax
import jax.numpy as jnp
from jax import lax
from jax.experimental import pallas as pl
from jax.experimental.pallas import tpu as pltpu
from jax.experimental.pallas import tpu_sc as plsc

N = 5000
NPAD = 5120
A = 8400
K = 80
NC = 2
NS = 16
L = 16
NH = NPAD // NC
KC = K // NS
CHUNK = NH // NS
NV1 = NH // L
NV2 = CHUNK // L

IDXW = (NH + 120) * 3
IDXOFF = (NH - 120) * 3

_MESH = plsc.VectorSubcoreMesh(core_axis_name="c", subcore_axis_name="s",
                               num_cores=NC, num_subcores=NS)


@functools.partial(
    pl.kernel,
    out_type=[
        jax.ShapeDtypeStruct((1, N, 4), jnp.float32),
        jax.ShapeDtypeStruct((1, N), jnp.float32),
        jax.ShapeDtypeStruct((1, N), jnp.int32),
    ],
    mesh=_MESH,
    compiler_params=pltpu.CompilerParams(
        needs_layout_passes=False,
        use_tc_tiling_on_sc=False,
    ),
    scratch_types=[
        pltpu.VMEM((IDXW,), jnp.int32),
        pltpu.VMEM((KC * A,), jnp.float32),
        pltpu.VMEM((NH,), jnp.float32),
        pltpu.VMEM((NH,), jnp.int32),
        pltpu.VMEM_SHARED((NS * NH,), jnp.float32),
        pltpu.VMEM_SHARED((NS * NH,), jnp.int32),
        pltpu.VMEM((NS * CHUNK,), jnp.float32),
        pltpu.VMEM((NS * CHUNK,), jnp.int32),
        pltpu.VMEM((4 * A,), jnp.float32),
        pltpu.VMEM((CHUNK, 4), jnp.float32),
        pltpu.VMEM((CHUNK,), jnp.float32),
        pltpu.VMEM((CHUNK,), jnp.int32),
        pltpu.SemaphoreType.DMA,
        pltpu.SemaphoreType.DMA,
    ],
)
def _sc_transform(idx_hbm, boxes_hbm, scores_hbm,
                  out_bbox, out_score, out_cls,
                  idx3_v, rows_v, pmax_v, pcls_v, shm, shc,
                  pbm, pbc, boxes_v, bbox_v, sco_v, clo_v,
                  sem_box, sem_p):
    cid = lax.axis_index("c")
    sid = lax.axis_index("s")
    lane3 = lax.broadcasted_iota(jnp.int32, (L,), 0) * 3

    box_cp = pltpu.make_async_copy(boxes_hbm, boxes_v, sem_box)
    box_cp.start()

    with jax.named_scope("p0_dma"):
        pltpu.sync_copy(idx_hbm.at[pl.ds(cid * IDXOFF, IDXW)], idx3_v)
        pltpu.sync_copy(scores_hbm.at[pl.ds(sid * (KC * A), KC * A)], rows_v)
    kbase = sid * KC
    poff = cid * 360 + 2

    _s1scope = jax.named_scope("p1_s1")
    _s1scope.__enter__()

    @plsc.parallel_loop(0, NV1, unroll=4)
    def s1_body(v):
        pos = jnp.minimum(lane3 + (v * (3 * L) + poff), IDXW - 1)
        iv = plsc.load_gather(idx3_v, [pos])
        m = plsc.load_gather(rows_v, [iv])
        cls = jnp.broadcast_to(kbase, (L,)).astype(jnp.int32)
        for k in range(1, KC):
            g = plsc.load_gather(rows_v, [iv + jnp.full((L,), k * A, jnp.int32)])
            upd = g > m
            m = jnp.where(upd, g, m)
            cls = jnp.where(upd,
                            jnp.broadcast_to(kbase + k, (L,)).astype(jnp.int32),
                            cls)
        pmax_v[pl.ds(v * L, L)] = m
        pcls_v[pl.ds(v * L, L)] = cls
    _s1scope.__exit__(None, None, None)
    with jax.named_scope("p2_pub"):
        pltpu.sync_copy(pmax_v, shm.at[pl.ds(sid * NH, NH)])
        pltpu.sync_copy(pcls_v, shc.at[pl.ds(sid * NH, NH)])
        plsc.subcore_barrier()

    base2 = sid * CHUNK
    with jax.named_scope("p3_read"):
        handles = []
        for t in range(NS):
            handles.append(pltpu.make_async_copy(
                shm.at[pl.ds(t * NH + base2, CHUNK)],
                pbm.at[pl.ds(t * CHUNK, CHUNK)], sem_p))
            handles.append(pltpu.make_async_copy(
                shc.at[pl.ds(t * NH + base2, CHUNK)],
                pbc.at[pl.ds(t * CHUNK, CHUNK)], sem_p))
        for h in handles:
            h.start()
        for h in handles:
            h.wait()
        box_cp.wait()
    lane = lax.broadcasted_iota(jnp.int32, (L,), 0)
    _s2scope = jax.named_scope("p4_s2")
    _s2scope.__enter__()

    @plsc.parallel_loop(0, NV2, unroll=2)
    def s2_body(v):
        off = v * L
        m = pbm[pl.ds(off, L)]
        cls = pbc[pl.ds(off, L)]
        for t in range(1, NS):
            mt = pbm[pl.ds(t * CHUNK + off, L)]
            ct = pbc[pl.ds(t * CHUNK + off, L)]
            upd = mt > m
            m = jnp.where(upd, mt, m)
            cls = jnp.where(upd, ct, cls)
        sco_v[pl.ds(off, L)] = m
        clo_v[pl.ds(off, L)] = cls

        pos = jnp.minimum(lane3 + ((base2 + off) * 3 + poff), IDXW - 1)
        iv = plsc.load_gather(idx3_v, [pos])
        cx = plsc.load_gather(boxes_v, [iv])
        cy = plsc.load_gather(boxes_v, [iv + jnp.full((L,), A, jnp.int32)])
        w = plsc.load_gather(boxes_v, [iv + jnp.full((L,), 2 * A, jnp.int32)])
        h = plsc.load_gather(boxes_v, [iv + jnp.full((L,), 3 * A, jnp.int32)])
        hw = 0.5 * w
        hh = 0.5 * h
        x1 = (cx - hw) / 640.0
        y1 = (cy - hh) / 480.0
        x2 = (cx + hw) / 640.0
        y2 = (cy + hh) / 480.0
        orow = off + lane
        zero = jnp.zeros((L,), jnp.int32)
        plsc.store_scatter(bbox_v, [orow, zero], x1)
        plsc.store_scatter(bbox_v, [orow, zero + 1], y1)
        plsc.store_scatter(bbox_v, [orow, zero + 2], x2)
        plsc.store_scatter(bbox_v, [orow, zero + 3], y2)

    _s2scope.__exit__(None, None, None)
    gbase = cid * NH + base2
    is_last = jnp.logical_and(cid == NC - 1, sid == NS - 1)
    last = N - (NC * NS - 1) * CHUNK

    @pl.when(jnp.logical_not(is_last))
    def _full():
        pltpu.sync_copy(sco_v, out_score.at[0, pl.ds(gbase, CHUNK)])
        pltpu.sync_copy(clo_v, out_cls.at[0, pl.ds(gbase, CHUNK)])
        pltpu.sync_copy(bbox_v, out_bbox.at[0, pl.ds(gbase, CHUNK), :])

    @pl.when(is_last)
    def _tail():
        pltpu.sync_copy(sco_v.at[pl.ds(0, last)],
                        out_score.at[0, pl.ds(N - last, last)])
        pltpu.sync_copy(clo_v.at[pl.ds(0, last)],
                        out_cls.at[0, pl.ds(N - last, last)])
        pltpu.sync_copy(bbox_v.at[pl.ds(0, last), :],
                        out_bbox.at[0, pl.ds(N - last, last), :])


def kernel(idxTensor, boxes, scores):
    idx_flat = idxTensor.reshape(N * 3)
    boxes_flat = boxes.reshape(4 * A)
    scores_flat = scores.reshape(K * A)
    bbox_xyxy, score_result, classes_result = _sc_transform(
        idx_flat, boxes_flat, scores_flat)
    num_dets = jnp.array(N, dtype=jnp.int32)
    return (bbox_xyxy, score_result, classes_result, num_dets)


# speedup vs baseline: 1.1256x; 1.1256x over previous
import functools

import jax
import jax.numpy as jnp
from jax import lax
from jax.experimental import pallas as pl
from jax.experimental.pallas import tpu as pltpu
from jax.experimental.pallas import tpu_sc as plsc

N = 5000
NPAD = 5120
A = 8400
K = 80
NC = 2
NS = 16
L = 16
NH = NPAD // NC
KC = K // NS
CHUNK = NH // NS
NV1 = NH // L
NV2 = CHUNK // L
MW = CHUNK * 3

_MESH = plsc.VectorSubcoreMesh(core_axis_name="c", subcore_axis_name="s",
                               num_cores=NC, num_subcores=NS)


@functools.partial(
    pl.kernel,
    out_type=[
        jax.ShapeDtypeStruct((1, N, 4), jnp.float32),
        jax.ShapeDtypeStruct((1, N), jnp.float32),
        jax.ShapeDtypeStruct((1, N), jnp.int32),
    ],
    mesh=_MESH,
    compiler_params=pltpu.CompilerParams(
        needs_layout_passes=False,
        use_tc_tiling_on_sc=False,
    ),
    scratch_types=[
        pltpu.VMEM((MW,), jnp.int32),
        pltpu.VMEM((CHUNK,), jnp.int32),
        pltpu.VMEM((NH,), jnp.int32),
        pltpu.VMEM((CHUNK * 4,), jnp.int32),
        pltpu.VMEM((CHUNK * 4,), jnp.float32),
        pltpu.VMEM((KC * A,), jnp.float32),
        pltpu.VMEM((NH,), jnp.float32),
        pltpu.VMEM((NH,), jnp.int32),
        pltpu.VMEM_SHARED((NH,), jnp.int32),
        pltpu.VMEM_SHARED((NS * NH,), jnp.float32),
        pltpu.VMEM_SHARED((NS * NH,), jnp.int32),
        pltpu.VMEM((NS * CHUNK,), jnp.float32),
        pltpu.VMEM((NS * CHUNK,), jnp.int32),
        pltpu.VMEM((CHUNK, 4), jnp.float32),
        pltpu.VMEM((CHUNK,), jnp.float32),
        pltpu.VMEM((CHUNK,), jnp.int32),
        pltpu.SemaphoreType.DMA,
        pltpu.SemaphoreType.DMA,
        pltpu.SemaphoreType.DMA,
    ],
)
def _sc_transform(idx_hbm, boxes_hbm, scores_hbm,
                  out_bbox, out_score, out_cls,
                  mini3_v, myiv_v, idx_v, ilist_v, braw_v, rows_v,
                  pmax_v, pcls_v, sh_idx, shm, shc, pbm, pbc,
                  bbox_v, sco_v, clo_v, sem_s, sem_box, sem_p):
    cid = lax.axis_index("c")
    sid = lax.axis_index("s")
    lane = lax.broadcasted_iota(jnp.int32, (L,), 0)
    lane3 = lane * 3
    gbase = cid * NH + sid * CHUNK
    start_row = jnp.minimum(gbase, N - CHUNK)
    pshift = (gbase - start_row) * 3 + 2

    sc_cp = pltpu.make_async_copy(
        scores_hbm.at[pl.ds(sid * (KC * A), KC * A)], rows_v, sem_s)
    sc_cp.start()
    with jax.named_scope("p0_idx"):
        pltpu.sync_copy(idx_hbm.at[pl.ds(start_row * 3, MW)], mini3_v)

        @plsc.parallel_loop(0, NV2, unroll=2)
        def extract(v):
            pos = jnp.minimum(lane3 + (v * (3 * L) + pshift), MW - 1)
            iv = plsc.load_gather(mini3_v, [pos])
            myiv_v[pl.ds(v * L, L)] = iv
            p4 = (v * L + lane) * 4
            plsc.store_scatter(ilist_v, [p4], iv)
            plsc.store_scatter(ilist_v, [p4 + 1], iv + jnp.full((L,), A, jnp.int32))
            plsc.store_scatter(ilist_v, [p4 + 2], iv + jnp.full((L,), 2 * A, jnp.int32))
            plsc.store_scatter(ilist_v, [p4 + 3], iv + jnp.full((L,), 3 * A, jnp.int32))

        pltpu.sync_copy(myiv_v, sh_idx.at[pl.ds(sid * CHUNK, CHUNK)])
        box_cp = pltpu.make_async_copy(boxes_hbm.at[ilist_v], braw_v, sem_box)
        box_cp.start()
        plsc.subcore_barrier()
        pltpu.sync_copy(sh_idx, idx_v)

    with jax.named_scope("p0_dma"):
        sc_cp.wait()

    kbase = sid * KC
    _s1scope = jax.named_scope("p1_s1")
    _s1scope.__enter__()

    @plsc.parallel_loop(0, NV1, unroll=4)
    def s1_body(v):
        iv = idx_v[pl.ds(v * L, L)]
        m = plsc.load_gather(rows_v, [iv])
        cls = jnp.broadcast_to(kbase, (L,)).astype(jnp.int32)
        for k in range(1, KC):
            g = plsc.load_gather(rows_v, [iv + jnp.full((L,), k * A, jnp.int32)])
            upd = g > m
            m = jnp.where(upd, g, m)
            cls = jnp.where(upd,
                            jnp.broadcast_to(kbase + k, (L,)).astype(jnp.int32),
                            cls)
        pmax_v[pl.ds(v * L, L)] = m
        pcls_v[pl.ds(v * L, L)] = cls
    _s1scope.__exit__(None, None, None)
    with jax.named_scope("p2_pub"):
        pltpu.sync_copy(pmax_v, shm.at[pl.ds(sid * NH, NH)])
        pltpu.sync_copy(pcls_v, shc.at[pl.ds(sid * NH, NH)])
        plsc.subcore_barrier()

    base2 = sid * CHUNK
    with jax.named_scope("p3_read"):
        handles = []
        for t in range(NS):
            handles.append(pltpu.make_async_copy(
                shm.at[pl.ds(t * NH + base2, CHUNK)],
                pbm.at[pl.ds(t * CHUNK, CHUNK)], sem_p))
            handles.append(pltpu.make_async_copy(
                shc.at[pl.ds(t * NH + base2, CHUNK)],
                pbc.at[pl.ds(t * CHUNK, CHUNK)], sem_p))
        for h in handles:
            h.start()
        for h in handles:
            h.wait()
        box_cp.wait()
    _s2scope = jax.named_scope("p4_s2")
    _s2scope.__enter__()

    @plsc.parallel_loop(0, NV2, unroll=2)
    def s2_body(v):
        off = v * L
        m = pbm[pl.ds(off, L)]
        cls = pbc[pl.ds(off, L)]
        for t in range(1, NS):
            mt = pbm[pl.ds(t * CHUNK + off, L)]
            ct = pbc[pl.ds(t * CHUNK + off, L)]
            upd = mt > m
            m = jnp.where(upd, mt, m)
            cls = jnp.where(upd, ct, cls)
        sco_v[pl.ds(off, L)] = m
        clo_v[pl.ds(off, L)] = cls

        p4 = (off + lane) * 4
        cx = plsc.load_gather(braw_v, [p4])
        cy = plsc.load_gather(braw_v, [p4 + 1])
        w = plsc.load_gather(braw_v, [p4 + 2])
        h = plsc.load_gather(braw_v, [p4 + 3])
        hw = 0.5 * w
        hh = 0.5 * h
        x1 = (cx - hw) / 640.0
        y1 = (cy - hh) / 480.0
        x2 = (cx + hw) / 640.0
        y2 = (cy + hh) / 480.0
        orow = off + lane
        zero = jnp.zeros((L,), jnp.int32)
        plsc.store_scatter(bbox_v, [orow, zero], x1)
        plsc.store_scatter(bbox_v, [orow, zero + 1], y1)
        plsc.store_scatter(bbox_v, [orow, zero + 2], x2)
        plsc.store_scatter(bbox_v, [orow, zero + 3], y2)
    _s2scope.__exit__(None, None, None)
    is_last = jnp.logical_and(cid == NC - 1, sid == NS - 1)
    last = N - (NC * NS - 1) * CHUNK

    @pl.when(jnp.logical_not(is_last))
    def _full():
        pltpu.sync_copy(sco_v, out_score.at[0, pl.ds(gbase, CHUNK)])
        pltpu.sync_copy(clo_v, out_cls.at[0, pl.ds(gbase, CHUNK)])
        pltpu.sync_copy(bbox_v, out_bbox.at[0, pl.ds(gbase, CHUNK), :])

    @pl.when(is_last)
    def _tail():
        pltpu.sync_copy(sco_v.at[pl.ds(0, last)],
                        out_score.at[0, pl.ds(N - last, last)])
        pltpu.sync_copy(clo_v.at[pl.ds(0, last)],
                        out_cls.at[0, pl.ds(N - last, last)])
        pltpu.sync_copy(bbox_v.at[pl.ds(0, last), :],
                        out_bbox.at[0, pl.ds(N - last, last), :])


def kernel(idxTensor, boxes, scores):
    idx_flat = idxTensor.reshape(N * 3)
    boxes_flat = boxes.reshape(4 * A)
    scores_flat = scores.reshape(K * A)
    bbox_xyxy, score_result, classes_result = _sc_transform(
        idx_flat, boxes_flat, scores_flat)
    num_dets = jnp.array(N, dtype=jnp.int32)
    return (bbox_xyxy, score_result, classes_result, num_dets)


# speedup vs baseline: 1.1286x; 1.0027x over previous
import functools

import jax
import jax.numpy as jnp
from jax import lax
from jax.experimental import pallas as pl
from jax.experimental.pallas import tpu as pltpu
from jax.experimental.pallas import tpu_sc as plsc

N = 5000
NPAD = 5120
A = 8400
K = 80
NC = 2
NS = 16
L = 16
NH = NPAD // NC
KC = K // NS
CHUNK = NH // NS
NV1 = NH // L
NV2 = CHUNK // L
MW = CHUNK * 3

_MESH = plsc.VectorSubcoreMesh(core_axis_name="c", subcore_axis_name="s",
                               num_cores=NC, num_subcores=NS)


@functools.partial(
    pl.kernel,
    out_type=[
        jax.ShapeDtypeStruct((1, N, 4), jnp.float32),
        jax.ShapeDtypeStruct((1, N), jnp.float32),
        jax.ShapeDtypeStruct((1, N), jnp.int32),
    ],
    mesh=_MESH,
    compiler_params=pltpu.CompilerParams(
        needs_layout_passes=False,
        use_tc_tiling_on_sc=False,
    ),
    scratch_types=[
        pltpu.VMEM((MW,), jnp.int32),
        pltpu.VMEM((CHUNK,), jnp.int32),
        pltpu.VMEM((NH,), jnp.int32),
        pltpu.VMEM((CHUNK * 4,), jnp.int32),
        pltpu.VMEM((CHUNK * 4,), jnp.float32),
        pltpu.VMEM((KC * A,), jnp.float32),
        pltpu.VMEM((NH,), jnp.float32),
        pltpu.VMEM((NH,), jnp.int32),
        pltpu.VMEM_SHARED((NH,), jnp.int32),
        pltpu.VMEM_SHARED((NS * NH,), jnp.float32),
        pltpu.VMEM_SHARED((NS * NH,), jnp.int32),
        pltpu.VMEM((NS * CHUNK,), jnp.float32),
        pltpu.VMEM((NS * CHUNK,), jnp.int32),
        pltpu.VMEM((CHUNK, 4), jnp.float32),
        pltpu.VMEM((CHUNK,), jnp.float32),
        pltpu.VMEM((CHUNK,), jnp.int32),
        pltpu.SemaphoreType.DMA,
        pltpu.SemaphoreType.DMA,
        pltpu.SemaphoreType.DMA,
    ],
)
def _sc_transform(idx_hbm, boxes_hbm, scores_hbm,
                  out_bbox, out_score, out_cls,
                  mini3_v, myiv_v, idx_v, ilist_v, braw_v, rows_v,
                  pmax_v, pcls_v, sh_idx, shm, shc, pbm, pbc,
                  bbox_v, sco_v, clo_v, sem_s, sem_box, sem_p):
    cid = lax.axis_index("c")
    sid = lax.axis_index("s")
    lane = lax.broadcasted_iota(jnp.int32, (L,), 0)
    lane3 = lane * 3
    gbase = cid * NH + sid * CHUNK
    start_row = jnp.minimum(gbase, N - CHUNK)
    pshift = (gbase - start_row) * 3 + 2

    sc_cp = pltpu.make_async_copy(
        scores_hbm.at[pl.ds(sid * (KC * A), KC * A)], rows_v, sem_s)
    sc_cp.start()
    with jax.named_scope("p0_idx"):
        pltpu.sync_copy(idx_hbm.at[pl.ds(start_row * 3, MW)], mini3_v)

        @plsc.parallel_loop(0, NV2, unroll=1)
        def extract(v):
            pos = jnp.minimum(lane3 + (v * (3 * L) + pshift), MW - 1)
            iv = plsc.load_gather(mini3_v, [pos])
            myiv_v[pl.ds(v * L, L)] = iv
            p4 = (v * L + lane) * 4
            plsc.store_scatter(ilist_v, [p4], iv)
            plsc.store_scatter(ilist_v, [p4 + 1], iv + jnp.full((L,), A, jnp.int32))
            plsc.store_scatter(ilist_v, [p4 + 2], iv + jnp.full((L,), 2 * A, jnp.int32))
            plsc.store_scatter(ilist_v, [p4 + 3], iv + jnp.full((L,), 3 * A, jnp.int32))

        pltpu.sync_copy(myiv_v, sh_idx.at[pl.ds(sid * CHUNK, CHUNK)])
        box_cp = pltpu.make_async_copy(boxes_hbm.at[ilist_v], braw_v, sem_box)
        box_cp.start()
        plsc.subcore_barrier()
        pltpu.sync_copy(sh_idx, idx_v)

    with jax.named_scope("p0_dma"):
        sc_cp.wait()

    kbase = sid * KC
    _s1scope = jax.named_scope("p1_s1")
    _s1scope.__enter__()

    @plsc.parallel_loop(0, NV1, unroll=2)
    def s1_body(v):
        iv = idx_v[pl.ds(v * L, L)]
        m = plsc.load_gather(rows_v, [iv])
        cls = jnp.broadcast_to(kbase, (L,)).astype(jnp.int32)
        for k in range(1, KC):
            g = plsc.load_gather(rows_v, [iv + jnp.full((L,), k * A, jnp.int32)])
            upd = g > m
            m = jnp.where(upd, g, m)
            cls = jnp.where(upd,
                            jnp.broadcast_to(kbase + k, (L,)).astype(jnp.int32),
                            cls)
        pmax_v[pl.ds(v * L, L)] = m
        pcls_v[pl.ds(v * L, L)] = cls
    _s1scope.__exit__(None, None, None)
    with jax.named_scope("p2_pub"):
        pltpu.sync_copy(pmax_v, shm.at[pl.ds(sid * NH, NH)])
        pltpu.sync_copy(pcls_v, shc.at[pl.ds(sid * NH, NH)])
        plsc.subcore_barrier()

    base2 = sid * CHUNK
    with jax.named_scope("p3_read"):
        handles = []
        for t in range(NS):
            handles.append(pltpu.make_async_copy(
                shm.at[pl.ds(t * NH + base2, CHUNK)],
                pbm.at[pl.ds(t * CHUNK, CHUNK)], sem_p))
            handles.append(pltpu.make_async_copy(
                shc.at[pl.ds(t * NH + base2, CHUNK)],
                pbc.at[pl.ds(t * CHUNK, CHUNK)], sem_p))
        for h in handles:
            h.start()
        for h in handles:
            h.wait()
        box_cp.wait()
    _s2scope = jax.named_scope("p4_s2")
    _s2scope.__enter__()

    @plsc.parallel_loop(0, NV2, unroll=1)
    def s2_body(v):
        off = v * L
        m = pbm[pl.ds(off, L)]
        cls = pbc[pl.ds(off, L)]
        for t in range(1, NS):
            mt = pbm[pl.ds(t * CHUNK + off, L)]
            ct = pbc[pl.ds(t * CHUNK + off, L)]
            upd = mt > m
            m = jnp.where(upd, mt, m)
            cls = jnp.where(upd, ct, cls)
        sco_v[pl.ds(off, L)] = m
        clo_v[pl.ds(off, L)] = cls

        p4 = (off + lane) * 4
        cx = plsc.load_gather(braw_v, [p4])
        cy = plsc.load_gather(braw_v, [p4 + 1])
        w = plsc.load_gather(braw_v, [p4 + 2])
        h = plsc.load_gather(braw_v, [p4 + 3])
        hw = 0.5 * w
        hh = 0.5 * h
        x1 = (cx - hw) / 640.0
        y1 = (cy - hh) / 480.0
        x2 = (cx + hw) / 640.0
        y2 = (cy + hh) / 480.0
        orow = off + lane
        zero = jnp.zeros((L,), jnp.int32)
        plsc.store_scatter(bbox_v, [orow, zero], x1)
        plsc.store_scatter(bbox_v, [orow, zero + 1], y1)
        plsc.store_scatter(bbox_v, [orow, zero + 2], x2)
        plsc.store_scatter(bbox_v, [orow, zero + 3], y2)
    _s2scope.__exit__(None, None, None)
    is_last = jnp.logical_and(cid == NC - 1, sid == NS - 1)
    last = N - (NC * NS - 1) * CHUNK

    @pl.when(jnp.logical_not(is_last))
    def _full():
        pltpu.sync_copy(sco_v, out_score.at[0, pl.ds(gbase, CHUNK)])
        pltpu.sync_copy(clo_v, out_cls.at[0, pl.ds(gbase, CHUNK)])
        pltpu.sync_copy(bbox_v, out_bbox.at[0, pl.ds(gbase, CHUNK), :])

    @pl.when(is_last)
    def _tail():
        pltpu.sync_copy(sco_v.at[pl.ds(0, last)],
                        out_score.at[0, pl.ds(N - last, last)])
        pltpu.sync_copy(clo_v.at[pl.ds(0, last)],
                        out_cls.at[0, pl.ds(N - last, last)])
        pltpu.sync_copy(bbox_v.at[pl.ds(0, last), :],
                        out_bbox.at[0, pl.ds(N - last, last), :])


def kernel(idxTensor, boxes, scores):
    idx_flat = idxTensor.reshape(N * 3)
    boxes_flat = boxes.reshape(4 * A)
    scores_flat = scores.reshape(K * A)
    bbox_xyxy, score_result, classes_result = _sc_transform(
        idx_flat, boxes_flat, scores_flat)
    num_dets = jnp.array(N, dtype=jnp.int32)
    return (bbox_xyxy, score_result, classes_result, num_dets)


# speedup vs baseline: 1.1769x; 1.0428x over previous
import functools

import jax
import jax.numpy as jnp
from jax import lax
from jax.experimental import pallas as pl
from jax.experimental.pallas import tpu as pltpu
from jax.experimental.pallas import tpu_sc as plsc

N = 5000
NPAD = 5120
A = 8400
K = 80
NC = 2
NS = 16
L = 16
NH = NPAD // NC
KC = K // NS
CHUNK = NH // NS
NV1 = NH // L
NV2 = CHUNK // L

_MESH = plsc.VectorSubcoreMesh(core_axis_name="c", subcore_axis_name="s",
                               num_cores=NC, num_subcores=NS)


@functools.partial(
    pl.kernel,
    out_type=[
        jax.ShapeDtypeStruct((1, N, 4), jnp.float32),
        jax.ShapeDtypeStruct((1, N), jnp.float32),
        jax.ShapeDtypeStruct((1, N), jnp.int32),
    ],
    mesh=_MESH,
    compiler_params=pltpu.CompilerParams(
        needs_layout_passes=False,
        use_tc_tiling_on_sc=False,
    ),
    scratch_types=[
        pltpu.VMEM((NH,), jnp.int32),
        pltpu.VMEM((CHUNK * 4,), jnp.int32),
        pltpu.VMEM((CHUNK * 4,), jnp.float32),
        pltpu.VMEM((KC * A,), jnp.float32),
        pltpu.VMEM((NH,), jnp.float32),
        pltpu.VMEM((NH,), jnp.int32),
        pltpu.VMEM_SHARED((NS * NH,), jnp.float32),
        pltpu.VMEM_SHARED((NS * NH,), jnp.int32),
        pltpu.VMEM((NS * CHUNK,), jnp.float32),
        pltpu.VMEM((NS * CHUNK,), jnp.int32),
        pltpu.VMEM((CHUNK, 4), jnp.float32),
        pltpu.VMEM((CHUNK,), jnp.float32),
        pltpu.VMEM((CHUNK,), jnp.int32),
        pltpu.SemaphoreType.DMA,
        pltpu.SemaphoreType.DMA,
        pltpu.SemaphoreType.DMA,
    ],
)
def _sc_transform(idx_hbm, boxes_hbm, scores_hbm,
                  out_bbox, out_score, out_cls,
                  idx_v, ilist_v, braw_v, rows_v, pmax_v, pcls_v,
                  shm, shc, pbm, pbc, bbox_v, sco_v, clo_v,
                  sem_s, sem_box, sem_p):
    cid = lax.axis_index("c")
    sid = lax.axis_index("s")
    lane = lax.broadcasted_iota(jnp.int32, (L,), 0)
    base2 = sid * CHUNK
    core_start = jnp.minimum(cid * NH, N - NH)
    doff = cid * NH - core_start

    sc_cp = pltpu.make_async_copy(
        scores_hbm.at[pl.ds(sid * (KC * A), KC * A)], rows_v, sem_s)
    sc_cp.start()
    pltpu.sync_copy(idx_hbm.at[pl.ds(core_start, NH)], idx_v)

    @plsc.parallel_loop(0, NV2, unroll=1)
    def build_ilist(v):
        pos = jnp.minimum(lane + (base2 + v * L) + doff, NH - 1)
        iv = plsc.load_gather(idx_v, [pos])
        p4 = (v * L + lane) * 4
        plsc.store_scatter(ilist_v, [p4], iv)
        plsc.store_scatter(ilist_v, [p4 + 1], iv + jnp.full((L,), A, jnp.int32))
        plsc.store_scatter(ilist_v, [p4 + 2], iv + jnp.full((L,), 2 * A, jnp.int32))
        plsc.store_scatter(ilist_v, [p4 + 3], iv + jnp.full((L,), 3 * A, jnp.int32))

    box_cp = pltpu.make_async_copy(boxes_hbm.at[ilist_v], braw_v, sem_box)
    box_cp.start()
    sc_cp.wait()

    kbase = sid * KC

    @plsc.parallel_loop(0, NV1, unroll=2)
    def s1_body(v):
        pos = jnp.minimum(lane + v * L + doff, NH - 1)
        iv = plsc.load_gather(idx_v, [pos])
        m = plsc.load_gather(rows_v, [iv])
        cls = jnp.broadcast_to(kbase, (L,)).astype(jnp.int32)
        for k in range(1, KC):
            g = plsc.load_gather(rows_v, [iv + jnp.full((L,), k * A, jnp.int32)])
            upd = g > m
            m = jnp.where(upd, g, m)
            cls = jnp.where(upd,
                            jnp.broadcast_to(kbase + k, (L,)).astype(jnp.int32),
                            cls)
        pmax_v[pl.ds(v * L, L)] = m
        pcls_v[pl.ds(v * L, L)] = cls

    pltpu.sync_copy(pmax_v, shm.at[pl.ds(sid * NH, NH)])
    pltpu.sync_copy(pcls_v, shc.at[pl.ds(sid * NH, NH)])
    plsc.subcore_barrier()

    handles = []
    for t in range(NS):
        handles.append(pltpu.make_async_copy(
            shm.at[pl.ds(t * NH + base2, CHUNK)],
            pbm.at[pl.ds(t * CHUNK, CHUNK)], sem_p))
        handles.append(pltpu.make_async_copy(
            shc.at[pl.ds(t * NH + base2, CHUNK)],
            pbc.at[pl.ds(t * CHUNK, CHUNK)], sem_p))
    for h in handles:
        h.start()
    for h in handles:
        h.wait()
    box_cp.wait()

    @plsc.parallel_loop(0, NV2, unroll=1)
    def s2_body(v):
        off = v * L
        m = pbm[pl.ds(off, L)]
        cls = pbc[pl.ds(off, L)]
        for t in range(1, NS):
            mt = pbm[pl.ds(t * CHUNK + off, L)]
            ct = pbc[pl.ds(t * CHUNK + off, L)]
            upd = mt > m
            m = jnp.where(upd, mt, m)
            cls = jnp.where(upd, ct, cls)
        sco_v[pl.ds(off, L)] = m
        clo_v[pl.ds(off, L)] = cls

        p4 = (off + lane) * 4
        cx = plsc.load_gather(braw_v, [p4])
        cy = plsc.load_gather(braw_v, [p4 + 1])
        w = plsc.load_gather(braw_v, [p4 + 2])
        h = plsc.load_gather(braw_v, [p4 + 3])
        hw = 0.5 * w
        hh = 0.5 * h
        x1 = (cx - hw) / 640.0
        y1 = (cy - hh) / 480.0
        x2 = (cx + hw) / 640.0
        y2 = (cy + hh) / 480.0
        orow = off + lane
        zero = jnp.zeros((L,), jnp.int32)
        plsc.store_scatter(bbox_v, [orow, zero], x1)
        plsc.store_scatter(bbox_v, [orow, zero + 1], y1)
        plsc.store_scatter(bbox_v, [orow, zero + 2], x2)
        plsc.store_scatter(bbox_v, [orow, zero + 3], y2)

    gbase = cid * NH + base2
    is_last = jnp.logical_and(cid == NC - 1, sid == NS - 1)
    last = N - (NC * NS - 1) * CHUNK

    @pl.when(jnp.logical_not(is_last))
    def _full():
        pltpu.sync_copy(sco_v, out_score.at[0, pl.ds(gbase, CHUNK)])
        pltpu.sync_copy(clo_v, out_cls.at[0, pl.ds(gbase, CHUNK)])
        pltpu.sync_copy(bbox_v, out_bbox.at[0, pl.ds(gbase, CHUNK), :])

    @pl.when(is_last)
    def _tail():
        pltpu.sync_copy(sco_v.at[pl.ds(0, last)],
                        out_score.at[0, pl.ds(N - last, last)])
        pltpu.sync_copy(clo_v.at[pl.ds(0, last)],
                        out_cls.at[0, pl.ds(N - last, last)])
        pltpu.sync_copy(bbox_v.at[pl.ds(0, last), :],
                        out_bbox.at[0, pl.ds(N - last, last), :])


def kernel(idxTensor, boxes, scores):
    idx_col = idxTensor[:, -1]
    boxes_flat = boxes.reshape(4 * A)
    scores_flat = scores.reshape(K * A)
    bbox_xyxy, score_result, classes_result = _sc_transform(
        idx_col, boxes_flat, scores_flat)
    num_dets = jnp.array(N, dtype=jnp.int32)
    return (bbox_xyxy, score_result, classes_result, num_dets)


# speedup vs baseline: 1.1810x; 1.0035x over previous
import functools

import jax
import jax.numpy as jnp
from jax import lax
from jax.experimental import pallas as pl
from jax.experimental.pallas import tpu as pltpu
from jax.experimental.pallas import tpu_sc as plsc

N = 5000
NPAD = 5120
A = 8400
K = 80
NC = 2
NS = 16
L = 16
NH = NPAD // NC
KC = K // NS
CHUNK = NH // NS
NV1 = NH // L
NV2 = CHUNK // L

_MESH = plsc.VectorSubcoreMesh(core_axis_name="c", subcore_axis_name="s",
                               num_cores=NC, num_subcores=NS)


@functools.partial(
    pl.kernel,
    out_type=[
        jax.ShapeDtypeStruct((1, N, 4), jnp.float32),
        jax.ShapeDtypeStruct((1, N), jnp.float32),
        jax.ShapeDtypeStruct((1, N), jnp.int32),
    ],
    mesh=_MESH,
    compiler_params=pltpu.CompilerParams(
        needs_layout_passes=False,
        use_tc_tiling_on_sc=False,
        skip_device_barrier=True,
    ),
    scratch_types=[
        pltpu.VMEM((NH,), jnp.int32),
        pltpu.VMEM((CHUNK * 4,), jnp.int32),
        pltpu.VMEM((CHUNK * 4,), jnp.float32),
        pltpu.VMEM((KC * A,), jnp.float32),
        pltpu.VMEM((NH,), jnp.float32),
        pltpu.VMEM((NH,), jnp.int32),
        pltpu.VMEM_SHARED((NS * NH,), jnp.float32),
        pltpu.VMEM_SHARED((NS * NH,), jnp.int32),
        pltpu.VMEM((NS * CHUNK,), jnp.float32),
        pltpu.VMEM((NS * CHUNK,), jnp.int32),
        pltpu.VMEM((CHUNK, 4), jnp.float32),
        pltpu.VMEM((CHUNK,), jnp.float32),
        pltpu.VMEM((CHUNK,), jnp.int32),
        pltpu.SemaphoreType.DMA,
        pltpu.SemaphoreType.DMA,
        pltpu.SemaphoreType.DMA,
    ],
)
def _sc_transform(idx_hbm, boxes_hbm, scores_hbm,
                  out_bbox, out_score, out_cls,
                  idx_v, ilist_v, braw_v, rows_v, pmax_v, pcls_v,
                  shm, shc, pbm, pbc, bbox_v, sco_v, clo_v,
                  sem_s, sem_box, sem_p):
    cid = lax.axis_index("c")
    sid = lax.axis_index("s")
    lane = lax.broadcasted_iota(jnp.int32, (L,), 0)
    base2 = sid * CHUNK
    core_start = jnp.minimum(cid * NH, N - NH)
    doff = cid * NH - core_start

    sc_cp = pltpu.make_async_copy(
        scores_hbm.at[pl.ds(sid * (KC * A), KC * A)], rows_v, sem_s)
    sc_cp.start()
    pltpu.sync_copy(idx_hbm.at[pl.ds(core_start, NH)], idx_v)

    @plsc.parallel_loop(0, NV2, unroll=1)
    def build_ilist(v):
        pos = jnp.minimum(lane + (base2 + v * L) + doff, NH - 1)
        iv = plsc.load_gather(idx_v, [pos])
        p4 = (v * L + lane) * 4
        plsc.store_scatter(ilist_v, [p4], iv)
        plsc.store_scatter(ilist_v, [p4 + 1], iv + jnp.full((L,), A, jnp.int32))
        plsc.store_scatter(ilist_v, [p4 + 2], iv + jnp.full((L,), 2 * A, jnp.int32))
        plsc.store_scatter(ilist_v, [p4 + 3], iv + jnp.full((L,), 3 * A, jnp.int32))

    box_cp = pltpu.make_async_copy(boxes_hbm.at[ilist_v], braw_v, sem_box)
    box_cp.start()
    sc_cp.wait()

    kbase = sid * KC

    @plsc.parallel_loop(0, NV1, unroll=2)
    def s1_body(v):
        pos = jnp.minimum(lane + v * L + doff, NH - 1)
        iv = plsc.load_gather(idx_v, [pos])
        m = plsc.load_gather(rows_v, [iv])
        cls = jnp.broadcast_to(kbase, (L,)).astype(jnp.int32)
        for k in range(1, KC):
            g = plsc.load_gather(rows_v, [iv + jnp.full((L,), k * A, jnp.int32)])
            upd = g > m
            m = jnp.where(upd, g, m)
            cls = jnp.where(upd,
                            jnp.broadcast_to(kbase + k, (L,)).astype(jnp.int32),
                            cls)
        pmax_v[pl.ds(v * L, L)] = m
        pcls_v[pl.ds(v * L, L)] = cls

    pltpu.sync_copy(pmax_v, shm.at[pl.ds(sid * NH, NH)])
    pltpu.sync_copy(pcls_v, shc.at[pl.ds(sid * NH, NH)])
    plsc.subcore_barrier()

    handles = []
    for t in range(NS):
        handles.append(pltpu.make_async_copy(
            shm.at[pl.ds(t * NH + base2, CHUNK)],
            pbm.at[pl.ds(t * CHUNK, CHUNK)], sem_p))
        handles.append(pltpu.make_async_copy(
            shc.at[pl.ds(t * NH + base2, CHUNK)],
            pbc.at[pl.ds(t * CHUNK, CHUNK)], sem_p))
    for h in handles:
        h.start()
    for h in handles:
        h.wait()
    box_cp.wait()

    @plsc.parallel_loop(0, NV2, unroll=1)
    def s2_body(v):
        off = v * L
        m = pbm[pl.ds(off, L)]
        cls = pbc[pl.ds(off, L)]
        for t in range(1, NS):
            mt = pbm[pl.ds(t * CHUNK + off, L)]
            ct = pbc[pl.ds(t * CHUNK + off, L)]
            upd = mt > m
            m = jnp.where(upd, mt, m)
            cls = jnp.where(upd, ct, cls)
        sco_v[pl.ds(off, L)] = m
        clo_v[pl.ds(off, L)] = cls

        p4 = (off + lane) * 4
        cx = plsc.load_gather(braw_v, [p4])
        cy = plsc.load_gather(braw_v, [p4 + 1])
        w = plsc.load_gather(braw_v, [p4 + 2])
        h = plsc.load_gather(braw_v, [p4 + 3])
        hw = 0.5 * w
        hh = 0.5 * h
        x1 = (cx - hw) / 640.0
        y1 = (cy - hh) / 480.0
        x2 = (cx + hw) / 640.0
        y2 = (cy + hh) / 480.0
        orow = off + lane
        zero = jnp.zeros((L,), jnp.int32)
        plsc.store_scatter(bbox_v, [orow, zero], x1)
        plsc.store_scatter(bbox_v, [orow, zero + 1], y1)
        plsc.store_scatter(bbox_v, [orow, zero + 2], x2)
        plsc.store_scatter(bbox_v, [orow, zero + 3], y2)

    gbase = cid * NH + base2
    is_last = jnp.logical_and(cid == NC - 1, sid == NS - 1)
    last = N - (NC * NS - 1) * CHUNK

    @pl.when(jnp.logical_not(is_last))
    def _full():
        pltpu.sync_copy(sco_v, out_score.at[0, pl.ds(gbase, CHUNK)])
        pltpu.sync_copy(clo_v, out_cls.at[0, pl.ds(gbase, CHUNK)])
        pltpu.sync_copy(bbox_v, out_bbox.at[0, pl.ds(gbase, CHUNK), :])

    @pl.when(is_last)
    def _tail():
        pltpu.sync_copy(sco_v.at[pl.ds(0, last)],
                        out_score.at[0, pl.ds(N - last, last)])
        pltpu.sync_copy(clo_v.at[pl.ds(0, last)],
                        out_cls.at[0, pl.ds(N - last, last)])
        pltpu.sync_copy(bbox_v.at[pl.ds(0, last), :],
                        out_bbox.at[0, pl.ds(N - last, last), :])


def kernel(idxTensor, boxes, scores):
    idx_col = idxTensor[:, -1]
    boxes_flat = boxes.reshape(4 * A)
    scores_flat = scores.reshape(K * A)
    bbox_xyxy, score_result, classes_result = _sc_transform(
        idx_col, boxes_flat, scores_flat)
    num_dets = jnp.array(N, dtype=jnp.int32)
    return (bbox_xyxy, score_result, classes_result, num_dets)
